# baseline (speedup 1.0000x reference)
import functools

import jax
import jax.numpy as jnp
from jax import lax
from jax.experimental import pallas as pl
from jax.experimental.pallas import tpu as pltpu
from jax.experimental.pallas import tpu_sc as plsc

NC, NS = 2, 16
NW = NC * NS

CHUNK = 800
NBUF = 2


def kernel(x, table):
    B, L = x.shape
    V, D = table.shape
    N = B * L
    assert N % NW == 0
    pw = N // NW
    assert pw % CHUNK == 0
    g_steps = pw // CHUNK
    assert g_steps % 2 == 0 and g_steps >= 4

    mesh = plsc.VectorSubcoreMesh(
        core_axis_name="c", subcore_axis_name="s",
        num_cores=NC, num_subcores=NS,
    )

    scratch = (
        [pltpu.VMEM((CHUNK,), jnp.int32) for _ in range(NBUF)]
        + [pltpu.VMEM((CHUNK, D), jnp.float32) for _ in range(NBUF)]
        + [pltpu.SemaphoreType.DMA for _ in range(3 * NBUF)]
    )

    @functools.partial(
        pl.kernel,
        out_type=jax.ShapeDtypeStruct((B, L, D), jnp.float32),
        mesh=mesh,
        scratch_types=scratch,
        compiler_params=pltpu.CompilerParams(use_tc_tiling_on_sc=False),
    )
    def emb(idx_hbm, tab_hbm, out_hbm,
            i0, i1, r0, r1, si0, si1, sg0, sg1, so0, so1):
        idx_v = [i0, i1]
        rows_v = [r0, r1]
        i_sem = [si0, si1]
        g_sem = [sg0, sg1]
        o_sem = [so0, so1]

        wid = lax.axis_index("s") * NC + lax.axis_index("c")
        base = wid * pw

        def idx_cp(g, b):
            return pltpu.make_async_copy(
                idx_hbm.at[pl.ds(base + g * CHUNK, CHUNK)], idx_v[b], i_sem[b])

        def gat_cp(b):
            return pltpu.make_async_copy(tab_hbm.at[idx_v[b]], rows_v[b], g_sem[b])

        def out_cps(g, b):
            brow0 = (base + g * CHUNK) // L
            return [pltpu.make_async_copy(
                        rows_v[b].at[pl.ds(j * L, L)],
                        out_hbm.at[brow0 + j], o_sem[b])
                    for j in range(CHUNK // L)]

        class _OutCp:
            def __init__(self, cps):
                self.cps = cps

            def start(self):
                for c in self.cps:
                    c.start()

            def wait(self):
                for c in self.cps:
                    c.wait()

        def out_cp(g, b):
            return _OutCp(out_cps(g, b))

        for b in range(NBUF):
            idx_cp(b, b).start()
        for b in range(NBUF):
            idx_cp(b, b).wait()
            gat_cp(b).start()
            gat_cp(b).wait()
            out_cp(b, b).start()
            idx_cp(b + NBUF, b).start()

        def pair(blk, carry):
            for b in range(NBUF):
                g = blk * NBUF + b
                idx_cp(g, b).wait()
                out_cp(g - NBUF, b).wait()
                gat_cp(b).start()
                gat_cp(b).wait()
                out_cp(g, b).start()
                idx_cp(g + NBUF, b).start()
            return carry

        lax.fori_loop(1, g_steps // NBUF - 1, pair, 0)

        for b in range(NBUF):
            g = g_steps - NBUF + b
            idx_cp(g, b).wait()
            out_cp(g - NBUF, b).wait()
            gat_cp(b).start()
            gat_cp(b).wait()
            out_cp(g, b).start()
        for b in range(NBUF):
            out_cp(g_steps - NBUF + b, b).wait()

    out = emb(x.reshape(N), table)
    return jnp.where((x >= 0)[:, :, None], out, 0.0)


# speedup vs baseline: 1.0038x; 1.0038x over previous
import functools

import jax
import jax.numpy as jnp
from jax import lax
from jax.experimental import pallas as pl
from jax.experimental.pallas import tpu as pltpu
from jax.experimental.pallas import tpu_sc as plsc

NC, NS = 2, 16
NW = NC * NS

HC = 400
NBUF = 2


def kernel(x, table):
    B, L = x.shape
    V, D = table.shape
    N = B * L
    half = N // 2
    assert half % NW == 0
    pw = half // NW
    assert pw % HC == 0
    g_steps = pw // HC
    assert g_steps % 2 == 0 and g_steps >= 4

    mesh = plsc.VectorSubcoreMesh(
        core_axis_name="c", subcore_axis_name="s",
        num_cores=NC, num_subcores=NS,
    )

    scratch = (
        [pltpu.VMEM((HC,), jnp.int32) for _ in range(2 * NBUF)]
        + [pltpu.VMEM((HC, D), jnp.float32) for _ in range(2 * NBUF)]
        + [pltpu.SemaphoreType.DMA for _ in range(3 * NBUF)]
    )

    @functools.partial(
        pl.kernel,
        out_type=jax.ShapeDtypeStruct((half, 2 * D), jnp.float32),
        mesh=mesh,
        scratch_types=scratch,
        compiler_params=pltpu.CompilerParams(use_tc_tiling_on_sc=False),
    )
    def emb(ide_hbm, ido_hbm, tab_hbm, out_hbm,
            ie0, ie1, io0, io1, re0, re1, ro0, ro1,
            si0, si1, sg0, sg1, so0, so1):
        ie_v = [ie0, ie1]
        io_v = [io0, io1]
        re_v = [re0, re1]
        ro_v = [ro0, ro1]
        i_sem = [si0, si1]
        g_sem = [sg0, sg1]
        o_sem = [so0, so1]

        wid = lax.axis_index("s") * NC + lax.axis_index("c")
        base = wid * pw

        def idx_cps(g, b):
            q0 = base + g * HC
            return [
                pltpu.make_async_copy(
                    ide_hbm.at[pl.ds(q0, HC)], ie_v[b], i_sem[b]),
                pltpu.make_async_copy(
                    ido_hbm.at[pl.ds(q0, HC)], io_v[b], i_sem[b]),
            ]

        def gat_cps(b):
            return [
                pltpu.make_async_copy(tab_hbm.at[ie_v[b]], re_v[b], g_sem[b]),
                pltpu.make_async_copy(tab_hbm.at[io_v[b]], ro_v[b], g_sem[b]),
            ]

        def out_cps(g, b):
            q0 = base + g * HC
            return [
                pltpu.make_async_copy(
                    re_v[b], out_hbm.at[pl.ds(q0, HC), pl.ds(0, D)], o_sem[b]),
                pltpu.make_async_copy(
                    ro_v[b], out_hbm.at[pl.ds(q0, HC), pl.ds(D, D)], o_sem[b]),
            ]

        def start(cps):
            for c in cps:
                c.start()

        def wait(cps):
            for c in cps:
                c.wait()

        for b in range(NBUF):
            start(idx_cps(b, b))
        for b in range(NBUF):
            wait(idx_cps(b, b))
            start(gat_cps(b))
            wait(gat_cps(b))
            start(out_cps(b, b))
            start(idx_cps(b + NBUF, b))

        def pair(blk, carry):
            for b in range(NBUF):
                g = blk * NBUF + b
                wait(idx_cps(g, b))
                wait(out_cps(g - NBUF, b))
                start(gat_cps(b))
                wait(gat_cps(b))
                start(out_cps(g, b))
                start(idx_cps(g + NBUF, b))
            return carry

        lax.fori_loop(1, g_steps // NBUF - 1, pair, 0)

        for b in range(NBUF):
            g = g_steps - NBUF + b
            wait(idx_cps(g, b))
            wait(out_cps(g - NBUF, b))
            start(gat_cps(b))
            wait(gat_cps(b))
            start(out_cps(g, b))
        for b in range(NBUF):
            wait(out_cps(g_steps - NBUF + b, b))

    x2 = x.reshape(half, 2)
    out2 = emb(x2[:, 0], x2[:, 1], table)
    return out2.reshape(B, L, D)


# speedup vs baseline: 1.7265x; 1.7199x over previous
import functools

import jax
import jax.numpy as jnp
from jax import lax
from jax.experimental import pallas as pl
from jax.experimental.pallas import tpu as pltpu
from jax.experimental.pallas import tpu_sc as plsc

NC, NS = 2, 16
NW = NC * NS

CHUNK = 400
NBUF = 2


def kernel(x, table):
    B, L = x.shape
    V, D = table.shape
    N = B * L
    W = 2 * D
    assert N % NW == 0
    pw = N // NW
    assert pw % CHUNK == 0 and CHUNK % L == 0
    g_steps = pw // CHUNK
    assert g_steps % 2 == 0 and g_steps >= 4
    rb = CHUNK // L

    mesh = plsc.VectorSubcoreMesh(
        core_axis_name="c", subcore_axis_name="s",
        num_cores=NC, num_subcores=NS,
    )

    scratch = (
        [pltpu.VMEM((CHUNK,), jnp.int32) for _ in range(NBUF)]
        + [pltpu.VMEM((CHUNK, W), jnp.float32) for _ in range(NBUF)]
        + [pltpu.SemaphoreType.DMA for _ in range(3 * NBUF)]
    )

    @functools.partial(
        pl.kernel,
        out_type=jax.ShapeDtypeStruct((B, L, W), jnp.float32),
        mesh=mesh,
        scratch_types=scratch,
    )
    def emb(idx_hbm, tab_hbm, out_hbm,
            i0, i1, r0, r1, si0, si1, sg0, sg1, so0, so1):
        idx_v = [i0, i1]
        rows_v = [r0, r1]
        i_sem = [si0, si1]
        g_sem = [sg0, sg1]
        o_sem = [so0, so1]

        wid = lax.axis_index("s") * NC + lax.axis_index("c")
        base = wid * pw

        def idx_cp(g, b):
            return pltpu.make_async_copy(
                idx_hbm.at[pl.ds(base + g * CHUNK, CHUNK)], idx_v[b], i_sem[b])

        def gat_cp(b):
            return pltpu.make_async_copy(tab_hbm.at[idx_v[b]], rows_v[b], g_sem[b])

        def out_cps(g, b):
            brow0 = (base + g * CHUNK) // L
            return [pltpu.make_async_copy(
                        rows_v[b].at[pl.ds(j * L, L)],
                        out_hbm.at[brow0 + j], o_sem[b])
                    for j in range(rb)]

        class _Multi:
            def __init__(self, cps):
                self.cps = cps

            def start(self):
                for c in self.cps:
                    c.start()

            def wait(self):
                for c in self.cps:
                    c.wait()

        def out_cp(g, b):
            return _Multi(out_cps(g, b))

        for b in range(NBUF):
            idx_cp(b, b).start()
        for b in range(NBUF):
            idx_cp(b, b).wait()
            gat_cp(b).start()
            gat_cp(b).wait()
            out_cp(b, b).start()
            idx_cp(b + NBUF, b).start()

        def pair(blk, carry):
            for b in range(NBUF):
                g = blk * NBUF + b
                idx_cp(g, b).wait()
                out_cp(g - NBUF, b).wait()
                gat_cp(b).start()
                gat_cp(b).wait()
                out_cp(g, b).start()
                idx_cp(g + NBUF, b).start()
            return carry

        lax.fori_loop(1, g_steps // NBUF - 1, pair, 0)

        for b in range(NBUF):
            g = g_steps - NBUF + b
            idx_cp(g, b).wait()
            out_cp(g - NBUF, b).wait()
            gat_cp(b).start()
            gat_cp(b).wait()
            out_cp(g, b).start()
        for b in range(NBUF):
            out_cp(g_steps - NBUF + b, b).wait()

    tab_p = jnp.pad(table, ((0, 0), (0, W - D)))
    out3 = emb(x.reshape(N), tab_p)
    return out3[:, :, :D]


# speedup vs baseline: 1.7323x; 1.0033x over previous
import functools

import jax
import jax.numpy as jnp
from jax import lax
from jax.experimental import pallas as pl
from jax.experimental.pallas import tpu as pltpu
from jax.experimental.pallas import tpu_sc as plsc

NC, NS = 2, 16
NW = NC * NS

CHUNK = 400
NIB = 4
NRB = 2


def kernel(x, table):
    B, L = x.shape
    V, D = table.shape
    N = B * L
    W = 2 * D
    assert N % NW == 0
    pw = N // NW
    assert pw % CHUNK == 0 and CHUNK % L == 0
    G = pw // CHUNK
    assert G % NIB == 0 and G >= 3 * NIB
    rb = CHUNK // L

    mesh = plsc.VectorSubcoreMesh(
        core_axis_name="c", subcore_axis_name="s",
        num_cores=NC, num_subcores=NS,
    )

    scratch = (
        [pltpu.VMEM((CHUNK,), jnp.int32) for _ in range(NIB)]
        + [pltpu.VMEM((CHUNK, W), jnp.float32) for _ in range(NRB)]
        + [pltpu.SemaphoreType.DMA for _ in range(NIB + 2 * NRB)]
    )

    @functools.partial(
        pl.kernel,
        out_type=jax.ShapeDtypeStruct((B, L, W), jnp.float32),
        mesh=mesh,
        scratch_types=scratch,
    )
    def emb(idx_hbm, tab_hbm, out_hbm,
            i0, i1, i2, i3, r0, r1,
            si0, si1, si2, si3, sg0, sg1, so0, so1):
        idx_v = [i0, i1, i2, i3]
        rows_v = [r0, r1]
        i_sem = [si0, si1, si2, si3]
        g_sem = [sg0, sg1]
        o_sem = [so0, so1]

        wid = lax.axis_index("s") * NC + lax.axis_index("c")
        base = wid * pw

        def idx_cp(g, s):
            return pltpu.make_async_copy(
                idx_hbm.at[pl.ds(base + g * CHUNK, CHUNK)], idx_v[s], i_sem[s])

        def gat_cp(s, b):
            return pltpu.make_async_copy(tab_hbm.at[idx_v[s]], rows_v[b], g_sem[b])

        def out_cps(g, b):
            brow0 = (base + g * CHUNK) // L
            return [pltpu.make_async_copy(
                        rows_v[b].at[pl.ds(j * L, L)],
                        out_hbm.at[brow0 + j], o_sem[b])
                    for j in range(rb)]

        def out_start(g, b):
            for c in out_cps(g, b):
                c.start()

        def out_wait(g, b):
            for c in out_cps(g, b):
                c.wait()

        for s in range(NIB):
            idx_cp(s, s).start()
        idx_cp(0, 0).wait()
        gat_cp(0, 0).start()
        idx_cp(1, 1).wait()
        gat_cp(1, 1).start()
        gat_cp(0, 0).wait()
        out_start(0, 0)
        idx_cp(NIB, 0).start()

        def body(g, s, b, refill):
            idx_cp(g, s).wait()
            out_wait(g - NRB, b)
            gat_cp(s, b).start()
            gat_cp(1 - b, 1 - b).wait()
            out_start(g - 1, 1 - b)
            if refill:
                idx_cp(g + NIB - 1, (s - 1) % NIB).start()

        body(2, 2, 0, True)
        body(3, 3, 1, True)

        def quad(blk, carry):
            for j in range(NIB):
                body(blk * NIB + j, j, j % NRB, True)
            return carry

        lax.fori_loop(1, G // NIB - 1, quad, 0)

        for j in range(NIB):
            body(G - NIB + j, j, j % NRB, j == 0)
        gat_cp(NIB - 1, (G - 1) % NRB).wait()
        out_start(G - 1, (G - 1) % NRB)
        out_wait(G - 2, (G - 2) % NRB)
        out_wait(G - 1, (G - 1) % NRB)

    tab_p = jnp.pad(table, ((0, 0), (0, W - D)))
    out3 = emb(x.reshape(N), tab_p)
    return out3[:, :, :D]


# speedup vs baseline: 1.7328x; 1.0003x over previous
import functools

import jax
import jax.numpy as jnp
from jax import lax
from jax.experimental import pallas as pl
from jax.experimental.pallas import tpu as pltpu
from jax.experimental.pallas import tpu_sc as plsc

NC, NS = 2, 16
NW = NC * NS

CHUNK = 400
NIB = 4
NRB = 2


def kernel(x, table):
    B, L = x.shape
    V, D = table.shape
    N = B * L
    W = 2 * D
    assert N % NW == 0
    pw = N // NW
    assert pw % CHUNK == 0 and CHUNK % L == 0
    G = pw // CHUNK
    assert G % NIB == 0 and G >= 3 * NIB
    rb = CHUNK // L

    mesh = plsc.VectorSubcoreMesh(
        core_axis_name="c", subcore_axis_name="s",
        num_cores=NC, num_subcores=NS,
    )

    scratch = (
        [pltpu.VMEM((CHUNK,), jnp.int32) for _ in range(NIB)]
        + [pltpu.VMEM((CHUNK, W), jnp.float32) for _ in range(NRB)]
        + [pltpu.SemaphoreType.DMA for _ in range(NIB + 2 * NRB)]
    )

    @functools.partial(
        pl.kernel,
        out_type=jax.ShapeDtypeStruct((B, L, W), jnp.float32),
        mesh=mesh,
        scratch_types=scratch,
    )
    def emb(idx_hbm, tab_hbm, out_hbm,
            i0, i1, i2, i3, r0, r1,
            si0, si1, si2, si3, sg0, sg1, so0, so1):
        idx_v = [i0, i1, i2, i3]
        rows_v = [r0, r1]
        i_sem = [si0, si1, si2, si3]
        g_sem = [sg0, sg1]
        o_sem = [so0, so1]

        wid = lax.axis_index("s") * NC + lax.axis_index("c")
        base = wid * pw

        def idx_cp(g, s):
            return pltpu.make_async_copy(
                idx_hbm.at[pl.ds(base + g * CHUNK, CHUNK)], idx_v[s], i_sem[s])

        def gat_cp(s, b):
            return pltpu.make_async_copy(tab_hbm.at[idx_v[s]], rows_v[b], g_sem[b])

        def out_cps(g, b):
            brow0 = (base + g * CHUNK) // L
            return [pltpu.make_async_copy(
                        rows_v[b].at[pl.ds(j * L, L)],
                        out_hbm.at[brow0 + j], o_sem[b])
                    for j in range(rb)]

        def out_start(g, b):
            for c in out_cps(g, b):
                c.start()

        def out_wait(g, b):
            for c in out_cps(g, b):
                c.wait()

        for s in range(NIB):
            idx_cp(s, s).start()
        idx_cp(0, 0).wait()
        gat_cp(0, 0).start()
        idx_cp(1, 1).wait()
        gat_cp(1, 1).start()
        gat_cp(0, 0).wait()
        out_start(0, 0)
        idx_cp(NIB, 0).start()

        def body(g, s, b, refill):
            idx_cp(g, s).wait()
            out_wait(g - NRB, b)
            gat_cp(s, b).start()
            gat_cp(1 - b, 1 - b).wait()
            out_start(g - 1, 1 - b)
            if refill:
                idx_cp(g + NIB - 1, (s - 1) % NIB).start()

        body(2, 2, 0, True)
        body(3, 3, 1, True)

        def quad(blk, carry):
            for j in range(NIB):
                body(blk * NIB + j, j, j % NRB, True)
            return carry

        lax.fori_loop(1, G // NIB - 1, quad, 0)

        for j in range(NIB):
            body(G - NIB + j, j, j % NRB, j == 0)
        gat_cp(NIB - 1, (G - 1) % NRB).wait()
        out_start(G - 1, (G - 1) % NRB)
        out_wait(G - 2, (G - 2) % NRB)
        out_wait(G - 1, (G - 1) % NRB)

    tab_p = jnp.concatenate(
        [table, jnp.zeros((V, W - D), jnp.float32)], axis=1)
    out3 = emb(x.reshape(N), tab_p)
    return out3[:, :, :D]
